# baseline (speedup 1.0000x reference)
import jax
import jax.numpy as jnp
from jax.experimental import pallas as pl
from jax.experimental.pallas import tpu as pltpu

B, N, H = 1024, 16, 512
H2 = H // 2
EDGE_DIM = 16
THR = 0.4
E_PER = 6 * (N - 1) + 6 * N
W_PER = E_PER // 2
NODES_PER = 3 * N

BB = 32
E_TOT = B * E_PER


def _main_kernel(zt_ref, za_ref, zf_ref, wq_ref, bq_ref, wk_ref, bk_ref,
                 emb_ref, nf_ref, ea_ref):
    zt = zt_ref[...]
    za = za_ref[...]
    zf = zf_ref[...]

    nodes = jnp.concatenate([zt, za, zf], axis=1)
    nf_ref[...] = nodes.reshape(BB * NODES_PER, H)

    q_in = jnp.concatenate([zt, za], axis=1).reshape(BB * 2 * N, H)
    k_in = jnp.concatenate([za, zf], axis=1).reshape(BB * 2 * N, H)
    q = jax.lax.dot(q_in.astype(jnp.bfloat16),
                    wq_ref[...].astype(jnp.bfloat16),
                    preferred_element_type=jnp.float32) + bq_ref[...]
    k = jax.lax.dot(k_in.astype(jnp.bfloat16),
                    wk_ref[...].astype(jnp.bfloat16),
                    preferred_element_type=jnp.float32) + bk_ref[...]
    q3 = q.reshape(BB, 2 * N, H2)
    k3 = k.reshape(BB, 2 * N, H2)
    qt, qa = q3[:, :N, :], q3[:, N:, :]
    ka, kf = k3[:, :N, :], k3[:, N:, :]
    qsel = jnp.concatenate([qt, qt, qa], axis=1).reshape(BB * 48, H2)
    ksel = jnp.concatenate([ka, kf, kf], axis=1).reshape(BB * 48, H2)
    ones16 = jnp.ones((H2, EDGE_DIM), jnp.float32)
    nq = (qsel * qsel) @ ones16
    nk = (ksel * ksel) @ ones16
    dots = (qsel * ksel) @ ones16
    cos = (dots * jax.lax.rsqrt(jnp.maximum(nq, 1e-24))
           * jax.lax.rsqrt(jnp.maximum(nk, 1e-24)))
    disc = 1.0 - jax.nn.sigmoid(cos)

    zero8 = jnp.zeros((1, 8), jnp.float32)
    col = jax.lax.broadcasted_iota(jnp.int32, (96, EDGE_DIM), 1)
    e3 = jnp.concatenate([emb_ref[3:4, :], zero8], axis=1)
    e4 = jnp.concatenate([emb_ref[4:5, :], zero8], axis=1)
    base3 = jnp.where(col < 8, e3, jnp.where(col == 11, 3.0 / 4.0, 0.0))
    base4 = jnp.where(col < 8, e4, jnp.where(col == 11, 4.0 / 4.0, 0.0))
    oh = ((jax.lax.broadcasted_iota(jnp.int32, (96, 48), 0) // 2)
          == jax.lax.broadcasted_iota(jnp.int32, (96, 48), 1)
          ).astype(jnp.float32)

    tr = jax.lax.broadcasted_iota(jnp.int32, (90, EDGE_DIM), 0)
    ta = jax.lax.broadcasted_iota(jnp.int32, (90, EDGE_DIM), 1)
    et = tr // 30
    e0 = jnp.concatenate([emb_ref[0:1, :], zero8], axis=1)
    e1 = jnp.concatenate([emb_ref[1:2, :], zero8], axis=1)
    e2 = jnp.concatenate([emb_ref[2:3, :], zero8], axis=1)
    embpart = jnp.where(et == 0, e0, jnp.where(et == 1, e1, e2))
    temporal = (jnp.where(ta < 8, embpart, 0.0)
                + jnp.where(ta == 9, 1.0 / N, 0.0)
                + jnp.where(ta == 10, 1.0, 0.0)
                + jnp.where(ta == 11, et.astype(jnp.float32) / 4.0, 0.0))

    for s in range(BB):
        d96 = jax.lax.dot(oh, disc[s * 48:(s + 1) * 48, :],
                          precision=jax.lax.Precision.HIGHEST)
        cross_s = jnp.where(col == 8, d96, jnp.where(d96 > THR, base4, base3))
        ea_ref[pl.ds(s * E_PER, 90), :] = temporal
        ea_ref[pl.ds(s * E_PER + 90, 96), :] = cross_s


def _index_kernel(ei_ref, bv_ref):
    def base_row(r):
        c = jax.lax.broadcasted_iota(jnp.int32, (1, E_PER), 1)
        p = c % 2
        t_val = (c // 30) * N + (c % 30) // 2 + jnp.where(r == 0, p, 1 - p)
        cc = c - 90
        m = cc // 32
        j = (cc % 32) // 2
        ao = jnp.where(m == 2, N, 0)
        bo = jnp.where(m == 0, N, 2 * N)
        c_val = j + jnp.where((p + r) % 2 == 0, ao, bo)
        return jnp.where(c < 90, t_val, c_val)

    offs = NODES_PER * jax.lax.broadcasted_iota(jnp.int32, (B, 1), 0)
    ei_ref[0, :, :] = base_row(0) + offs
    ei_ref[1, :, :] = base_row(1) + offs
    bv_ref[...] = jax.lax.broadcasted_iota(jnp.int32, (B, NODES_PER), 0)


def kernel(z_text_segs, z_audio_segs, z_facial_segs, Wq, bq, Wk, bk, emb):
    nf, ea = pl.pallas_call(
        _main_kernel,
        grid=(B // BB,),
        in_specs=[
            pl.BlockSpec((BB, N, H), lambda i: (i, 0, 0)),
            pl.BlockSpec((BB, N, H), lambda i: (i, 0, 0)),
            pl.BlockSpec((BB, N, H), lambda i: (i, 0, 0)),
            pl.BlockSpec((H, H2), lambda i: (0, 0)),
            pl.BlockSpec((1, H2), lambda i: (0, 0)),
            pl.BlockSpec((H, H2), lambda i: (0, 0)),
            pl.BlockSpec((1, H2), lambda i: (0, 0)),
            pl.BlockSpec((5, 8), lambda i: (0, 0)),
        ],
        out_specs=[
            pl.BlockSpec((BB * NODES_PER, H), lambda i: (i, 0)),
            pl.BlockSpec((BB * E_PER, EDGE_DIM), lambda i: (i, 0)),
        ],
        out_shape=[
            jax.ShapeDtypeStruct((B * NODES_PER, H), jnp.float32),
            jax.ShapeDtypeStruct((E_TOT, EDGE_DIM), jnp.float32),
        ],
        compiler_params=pltpu.CompilerParams(
            dimension_semantics=("arbitrary",),
        ),
    )(z_text_segs, z_audio_segs, z_facial_segs, Wq, bq.reshape(1, H2),
      Wk, bk.reshape(1, H2), emb)
    ei, bv = pl.pallas_call(
        _index_kernel,
        out_shape=[
            jax.ShapeDtypeStruct((2, B, E_PER), jnp.int32),
            jax.ShapeDtypeStruct((B, NODES_PER), jnp.int32),
        ],
    )()
    return nf, ei.reshape(2, E_TOT), ea, bv.reshape(B * NODES_PER)


# speedup vs baseline: 1.1426x; 1.1426x over previous
import jax
import jax.numpy as jnp
from jax.experimental import pallas as pl
from jax.experimental.pallas import tpu as pltpu

B, N, H = 1024, 16, 512
H2 = H // 2
EDGE_DIM = 16
THR = 0.4
E_PER = 6 * (N - 1) + 6 * N
W_PER = E_PER // 2
NODES_PER = 3 * N

BB = 64
E_TOT = B * E_PER


def _main_kernel(zt_ref, za_ref, zf_ref, wq_ref, bq_ref, wk_ref, bk_ref,
                 emb_ref, nf_ref, ea_ref):
    zt = zt_ref[...]
    za = za_ref[...]
    zf = zf_ref[...]

    nodes = jnp.concatenate([zt, za, zf], axis=1)
    nf_ref[...] = nodes.reshape(BB * NODES_PER, H)

    q_in = jnp.concatenate([zt, za], axis=1).reshape(BB * 2 * N, H)
    k_in = jnp.concatenate([za, zf], axis=1).reshape(BB * 2 * N, H)
    q = jax.lax.dot(q_in.astype(jnp.bfloat16),
                    wq_ref[...].astype(jnp.bfloat16),
                    preferred_element_type=jnp.float32) + bq_ref[...]
    k = jax.lax.dot(k_in.astype(jnp.bfloat16),
                    wk_ref[...].astype(jnp.bfloat16),
                    preferred_element_type=jnp.float32) + bk_ref[...]
    q3 = q.reshape(BB, 2 * N, H2)
    k3 = k.reshape(BB, 2 * N, H2)
    qt, qa = q3[:, :N, :], q3[:, N:, :]
    ka, kf = k3[:, :N, :], k3[:, N:, :]
    qsel = jnp.concatenate([qt, qt, qa], axis=1).reshape(BB * 48, H2)
    ksel = jnp.concatenate([ka, kf, kf], axis=1).reshape(BB * 48, H2)
    ones16 = jnp.ones((H2, EDGE_DIM), jnp.float32)
    nq = (qsel * qsel) @ ones16
    nk = (ksel * ksel) @ ones16
    dots = (qsel * ksel) @ ones16
    cos = (dots * jax.lax.rsqrt(jnp.maximum(nq, 1e-24))
           * jax.lax.rsqrt(jnp.maximum(nk, 1e-24)))
    disc = 1.0 - jax.nn.sigmoid(cos)

    zero8 = jnp.zeros((1, 8), jnp.float32)
    col = jax.lax.broadcasted_iota(jnp.int32, (96, EDGE_DIM), 1)
    e3 = jnp.concatenate([emb_ref[3:4, :], zero8], axis=1)
    e4 = jnp.concatenate([emb_ref[4:5, :], zero8], axis=1)
    base3 = jnp.where(col < 8, e3, jnp.where(col == 11, 3.0 / 4.0, 0.0))
    base4 = jnp.where(col < 8, e4, jnp.where(col == 11, 4.0 / 4.0, 0.0))
    oh = ((jax.lax.broadcasted_iota(jnp.int32, (96, 48), 0) // 2)
          == jax.lax.broadcasted_iota(jnp.int32, (96, 48), 1)
          ).astype(jnp.float32)

    tr = jax.lax.broadcasted_iota(jnp.int32, (90, EDGE_DIM), 0)
    ta = jax.lax.broadcasted_iota(jnp.int32, (90, EDGE_DIM), 1)
    et = tr // 30
    e0 = jnp.concatenate([emb_ref[0:1, :], zero8], axis=1)
    e1 = jnp.concatenate([emb_ref[1:2, :], zero8], axis=1)
    e2 = jnp.concatenate([emb_ref[2:3, :], zero8], axis=1)
    embpart = jnp.where(et == 0, e0, jnp.where(et == 1, e1, e2))
    temporal = (jnp.where(ta < 8, embpart, 0.0)
                + jnp.where(ta == 9, 1.0 / N, 0.0)
                + jnp.where(ta == 10, 1.0, 0.0)
                + jnp.where(ta == 11, et.astype(jnp.float32) / 4.0, 0.0))

    for s in range(BB):
        d96 = jax.lax.dot(oh, disc[s * 48:(s + 1) * 48, :])
        cross_s = jnp.where(col == 8, d96, jnp.where(d96 > THR, base4, base3))
        ea_ref[pl.ds(s * E_PER, 90), :] = temporal
        ea_ref[pl.ds(s * E_PER + 90, 96), :] = cross_s


def _index_kernel(ei_ref, bv_ref):
    def base_row(r):
        c = jax.lax.broadcasted_iota(jnp.int32, (1, E_PER), 1)
        p = c % 2
        t_val = (c // 30) * N + (c % 30) // 2 + jnp.where(r == 0, p, 1 - p)
        cc = c - 90
        m = cc // 32
        j = (cc % 32) // 2
        ao = jnp.where(m == 2, N, 0)
        bo = jnp.where(m == 0, N, 2 * N)
        c_val = j + jnp.where((p + r) % 2 == 0, ao, bo)
        return jnp.where(c < 90, t_val, c_val)

    offs = NODES_PER * jax.lax.broadcasted_iota(jnp.int32, (B, 1), 0)
    ei_ref[0, :, :] = base_row(0) + offs
    ei_ref[1, :, :] = base_row(1) + offs
    bv_ref[...] = jax.lax.broadcasted_iota(jnp.int32, (B, NODES_PER), 0)


def kernel(z_text_segs, z_audio_segs, z_facial_segs, Wq, bq, Wk, bk, emb):
    nf, ea = pl.pallas_call(
        _main_kernel,
        grid=(B // BB,),
        in_specs=[
            pl.BlockSpec((BB, N, H), lambda i: (i, 0, 0)),
            pl.BlockSpec((BB, N, H), lambda i: (i, 0, 0)),
            pl.BlockSpec((BB, N, H), lambda i: (i, 0, 0)),
            pl.BlockSpec((H, H2), lambda i: (0, 0)),
            pl.BlockSpec((1, H2), lambda i: (0, 0)),
            pl.BlockSpec((H, H2), lambda i: (0, 0)),
            pl.BlockSpec((1, H2), lambda i: (0, 0)),
            pl.BlockSpec((5, 8), lambda i: (0, 0)),
        ],
        out_specs=[
            pl.BlockSpec((BB * NODES_PER, H), lambda i: (i, 0)),
            pl.BlockSpec((BB * E_PER, EDGE_DIM), lambda i: (i, 0)),
        ],
        out_shape=[
            jax.ShapeDtypeStruct((B * NODES_PER, H), jnp.float32),
            jax.ShapeDtypeStruct((E_TOT, EDGE_DIM), jnp.float32),
        ],
        compiler_params=pltpu.CompilerParams(
            dimension_semantics=("arbitrary",),
        ),
    )(z_text_segs, z_audio_segs, z_facial_segs, Wq, bq.reshape(1, H2),
      Wk, bk.reshape(1, H2), emb)
    ei, bv = pl.pallas_call(
        _index_kernel,
        out_shape=[
            jax.ShapeDtypeStruct((2, B, E_PER), jnp.int32),
            jax.ShapeDtypeStruct((B, NODES_PER), jnp.int32),
        ],
    )()
    return nf, ei.reshape(2, E_TOT), ea, bv.reshape(B * NODES_PER)
